# Initial kernel scaffold; baseline (speedup 1.0000x reference)
import jax
import jax.numpy as jnp
from jax.experimental import pallas as pl


def kernel(x, A, p):
    raise NotImplementedError("write your pallas kernel here")


# speedup vs baseline: 1.3007x; 1.3007x over previous
import functools

import jax
import jax.numpy as jnp
from jax.experimental import pallas as pl
from jax.experimental.pallas import tpu as pltpu

_HI = jax.lax.Precision.HIGHEST


def _b16(a):
    return a.astype(jnp.bfloat16).astype(jnp.float32)


def _invn_body(p_ref, o_ref):
    pb = p_ref[...]
    o_ref[0, 0] = jax.lax.rsqrt(jnp.maximum(jnp.sum(pb * pb), 1e-12))


def _score_body(invn_ref, p_ref, x_ref, y_ref):
    ctb = pl.program_id(1)

    @pl.when(ctb == 0)
    def _init():
        y_ref[...] = jnp.zeros_like(y_ref)

    xb = _b16(x_ref[0])
    pnb = _b16(p_ref[...] * invn_ref[0, 0])
    y_ref[0] += jnp.sum(xb * pnb, axis=0, keepdims=True)


def _select_body(y_ref, yt_ref, rank_ref, sigy_ref):
    n = pl.program_id(0)
    nn = pl.num_programs(0)
    v = y_ref.shape[2]

    yrow = y_ref[0]
    en = (jax.lax.broadcasted_iota(jnp.int32, (nn, 1), 0) == n).astype(jnp.float32)
    ycol = jax.lax.dot_general(yt_ref[...], en, (((1,), (0,)), ((), ())),
                               preferred_element_type=jnp.float32, precision=_HI)

    gt = ycol > yrow
    eq = ycol == yrow
    iw = jax.lax.broadcasted_iota(jnp.int32, (v, v), 0)
    iv = jax.lax.broadcasted_iota(jnp.int32, (v, v), 1)
    m = jnp.logical_or(gt, jnp.logical_and(eq, iw < iv))
    rank_ref[0] = jnp.sum(m.astype(jnp.float32), axis=0, keepdims=True)
    sigy_ref[0] = jax.nn.sigmoid(yrow)


def _xgather_body(rank_ref, sigy_ref, x_ref, o_ref, *, kk):
    v = rank_ref.shape[2]
    rkrow = rank_ref[0]
    sgrow = sigy_ref[0]
    jj = jax.lax.broadcasted_iota(jnp.int32, (kk, v), 0).astype(jnp.float32)
    sel = jj == rkrow
    wgt = jnp.where(sel, sgrow, 0.0)
    o_ref[0] = jax.lax.dot_general(x_ref[0], wgt, (((1,), (1,)), ((), ())),
                                   preferred_element_type=jnp.float32, precision=_HI)


def _a_body(rank_ref, a_ref, o_ref, *, kk):
    v = rank_ref.shape[2]
    rkrow = rank_ref[0]
    jj = jax.lax.broadcasted_iota(jnp.int32, (kk, v), 0).astype(jnp.float32)
    st = (jj == rkrow).astype(jnp.float32)
    a = _b16(a_ref[0])
    u = jax.lax.dot_general(st, a, (((1,), (0,)), ((), ())),
                            preferred_element_type=jnp.float32, precision=_HI)
    asel = jax.lax.dot_general(a, st, (((1,), (1,)), ((), ())),
                               preferred_element_type=jnp.float32, precision=_HI)
    o_ref[0] = jax.lax.dot_general(u, asel, (((1,), (0,)), ((), ())),
                                   preferred_element_type=jnp.float32, precision=_HI)


@jax.jit
def kernel(x, A, p):
    n, c, t, v = x.shape
    ct = c * t
    kk = v // 2
    nk = A.shape[1]

    xr = x.reshape(n, ct, v)
    pc = p.reshape(ct, 1)

    invn = pl.pallas_call(
        _invn_body,
        out_specs=pl.BlockSpec(memory_space=pltpu.SMEM),
        out_shape=jax.ShapeDtypeStruct((1, 1), jnp.float32),
    )(p.reshape(64, ct // 64))

    blk1 = 2048
    y3 = pl.pallas_call(
        _score_body,
        grid=(n, ct // blk1),
        in_specs=[
            pl.BlockSpec(memory_space=pltpu.SMEM),
            pl.BlockSpec((blk1, 1), lambda i, j: (j, 0)),
            pl.BlockSpec((1, blk1, v), lambda i, j: (i, j, 0)),
        ],
        out_specs=pl.BlockSpec((1, 1, v), lambda i, j: (i, 0, 0)),
        out_shape=jax.ShapeDtypeStruct((n, 1, v), jnp.float32),
    )(invn, pc, xr)

    yt = y3.reshape(n, v).T

    rank3, sigy3 = pl.pallas_call(
        _select_body,
        grid=(n,),
        in_specs=[
            pl.BlockSpec((1, 1, v), lambda i: (i, 0, 0)),
            pl.BlockSpec((v, n), lambda i: (0, 0)),
        ],
        out_specs=[
            pl.BlockSpec((1, 1, v), lambda i: (i, 0, 0)),
            pl.BlockSpec((1, 1, v), lambda i: (i, 0, 0)),
        ],
        out_shape=[
            jax.ShapeDtypeStruct((n, 1, v), jnp.float32),
            jax.ShapeDtypeStruct((n, 1, v), jnp.float32),
        ],
    )(y3, yt)

    blk2 = 2048
    xo = pl.pallas_call(
        functools.partial(_xgather_body, kk=kk),
        grid=(n, ct // blk2),
        in_specs=[
            pl.BlockSpec((1, 1, v), lambda i, j: (i, 0, 0)),
            pl.BlockSpec((1, 1, v), lambda i, j: (i, 0, 0)),
            pl.BlockSpec((1, blk2, v), lambda i, j: (i, j, 0)),
        ],
        out_specs=pl.BlockSpec((1, blk2, kk), lambda i, j: (i, j, 0)),
        out_shape=jax.ShapeDtypeStruct((n, ct, kk), jnp.float32),
    )(rank3, sigy3, xr)
    x_out = xo.reshape(n, c, t, kk)

    ar = A.reshape(n * nk, v, v)
    ao = pl.pallas_call(
        functools.partial(_a_body, kk=kk),
        grid=(n * nk,),
        in_specs=[
            pl.BlockSpec((1, 1, v), lambda g: (g // nk, 0, 0)),
            pl.BlockSpec((1, v, v), lambda g: (g, 0, 0)),
        ],
        out_specs=pl.BlockSpec((1, kk, kk), lambda g: (g, 0, 0)),
        out_shape=jax.ShapeDtypeStruct((n * nk, kk, kk), jnp.float32),
    )(rank3, ar)
    A_out = ao.reshape(n, nk, kk, kk)

    return x_out, A_out


# speedup vs baseline: 1.8971x; 1.4585x over previous
import functools

import jax
import jax.numpy as jnp
from jax.experimental import pallas as pl
from jax.experimental.pallas import tpu as pltpu

_HI = jax.lax.Precision.HIGHEST


def _b16(a):
    return a.astype(jnp.bfloat16).astype(jnp.float32)


def _invn_body(p_ref, o_ref):
    pb = p_ref[...]
    o_ref[0, 0] = jax.lax.rsqrt(jnp.maximum(jnp.sum(pb * pb), 1e-12))


def _score_body(invn_ref, p_ref, x_ref, y_ref):
    ctb = pl.program_id(1)

    @pl.when(ctb == 0)
    def _init():
        y_ref[...] = jnp.zeros_like(y_ref)

    xb = _b16(x_ref[0])
    pnb = _b16(p_ref[...] * invn_ref[0, 0])
    y_ref[0] += jnp.sum(xb * pnb, axis=0, keepdims=True)


def _select_body(y_ref, yt_ref, rank_ref, sigy_ref):
    n = pl.program_id(0)
    nn = pl.num_programs(0)
    v = y_ref.shape[2]

    yrow = y_ref[0]
    en = (jax.lax.broadcasted_iota(jnp.int32, (nn, 1), 0) == n).astype(jnp.float32)
    ycol = jax.lax.dot_general(yt_ref[...], en, (((1,), (0,)), ((), ())),
                               preferred_element_type=jnp.float32, precision=_HI)

    gt = ycol > yrow
    eq = ycol == yrow
    iw = jax.lax.broadcasted_iota(jnp.int32, (v, v), 0)
    iv = jax.lax.broadcasted_iota(jnp.int32, (v, v), 1)
    m = jnp.logical_or(gt, jnp.logical_and(eq, iw < iv))
    rank_ref[0] = jnp.sum(m.astype(jnp.float32), axis=0, keepdims=True)
    sigy_ref[0] = jax.nn.sigmoid(yrow)


def _xgather_body(rank_ref, sigy_ref, x_ref, o_ref, *, kk):
    v = rank_ref.shape[2]
    rkrow = rank_ref[0]
    sgrow = sigy_ref[0]
    jj = jax.lax.broadcasted_iota(jnp.int32, (kk, v), 0).astype(jnp.float32)
    st = (jj == rkrow).astype(jnp.float32)
    sgate = jax.lax.dot_general(sgrow, st, (((1,), (1,)), ((), ())),
                                preferred_element_type=jnp.float32, precision=_HI)
    xf = x_ref[0]
    xhi = xf.astype(jnp.bfloat16)
    xlo = (xf - xhi.astype(jnp.float32)).astype(jnp.bfloat16)
    stb = st.astype(jnp.bfloat16)
    nt = (((1,), (1,)), ((), ()))
    acc = jax.lax.dot_general(xhi, stb, nt, preferred_element_type=jnp.float32)
    acc += jax.lax.dot_general(xlo, stb, nt, preferred_element_type=jnp.float32)
    o_ref[0] = acc * sgate


def _a_body(rank_ref, a_ref, o_ref, *, kk):
    v = rank_ref.shape[2]
    rkrow = rank_ref[0]
    jj = jax.lax.broadcasted_iota(jnp.int32, (kk, v), 0).astype(jnp.float32)
    st = (jj == rkrow).astype(jnp.float32)
    a = _b16(a_ref[0])
    u = jax.lax.dot_general(st, a, (((1,), (0,)), ((), ())),
                            preferred_element_type=jnp.float32)
    asel = jax.lax.dot_general(a, st, (((1,), (1,)), ((), ())),
                               preferred_element_type=jnp.float32)
    o_ref[0] = jax.lax.dot_general(u, asel, (((1,), (0,)), ((), ())),
                                   preferred_element_type=jnp.float32)


@jax.jit
def kernel(x, A, p):
    n, c, t, v = x.shape
    ct = c * t
    kk = v // 2
    nk = A.shape[1]

    xr = x.reshape(n, ct, v)
    pc = p.reshape(ct, 1)

    invn = pl.pallas_call(
        _invn_body,
        out_specs=pl.BlockSpec(memory_space=pltpu.SMEM),
        out_shape=jax.ShapeDtypeStruct((1, 1), jnp.float32),
    )(p.reshape(64, ct // 64))

    blk1 = 4096
    y3 = pl.pallas_call(
        _score_body,
        grid=(n, ct // blk1),
        in_specs=[
            pl.BlockSpec(memory_space=pltpu.SMEM),
            pl.BlockSpec((blk1, 1), lambda i, j: (j, 0)),
            pl.BlockSpec((1, blk1, v), lambda i, j: (i, j, 0)),
        ],
        out_specs=pl.BlockSpec((1, 1, v), lambda i, j: (i, 0, 0)),
        out_shape=jax.ShapeDtypeStruct((n, 1, v), jnp.float32),
    )(invn, pc, xr)

    yt = y3.reshape(n, v).T

    rank3, sigy3 = pl.pallas_call(
        _select_body,
        grid=(n,),
        in_specs=[
            pl.BlockSpec((1, 1, v), lambda i: (i, 0, 0)),
            pl.BlockSpec((v, n), lambda i: (0, 0)),
        ],
        out_specs=[
            pl.BlockSpec((1, 1, v), lambda i: (i, 0, 0)),
            pl.BlockSpec((1, 1, v), lambda i: (i, 0, 0)),
        ],
        out_shape=[
            jax.ShapeDtypeStruct((n, 1, v), jnp.float32),
            jax.ShapeDtypeStruct((n, 1, v), jnp.float32),
        ],
    )(y3, yt)

    blk2 = 4096
    xo = pl.pallas_call(
        functools.partial(_xgather_body, kk=kk),
        grid=(n, ct // blk2),
        in_specs=[
            pl.BlockSpec((1, 1, v), lambda i, j: (i, 0, 0)),
            pl.BlockSpec((1, 1, v), lambda i, j: (i, 0, 0)),
            pl.BlockSpec((1, blk2, v), lambda i, j: (i, j, 0)),
        ],
        out_specs=pl.BlockSpec((1, blk2, kk), lambda i, j: (i, j, 0)),
        out_shape=jax.ShapeDtypeStruct((n, ct, kk), jnp.float32),
    )(rank3, sigy3, xr)
    x_out = xo.reshape(n, c, t, kk)

    ar = A.reshape(n * nk, v, v)
    ao = pl.pallas_call(
        functools.partial(_a_body, kk=kk),
        grid=(n * nk,),
        in_specs=[
            pl.BlockSpec((1, 1, v), lambda g: (g // nk, 0, 0)),
            pl.BlockSpec((1, v, v), lambda g: (g, 0, 0)),
        ],
        out_specs=pl.BlockSpec((1, kk, kk), lambda g: (g, 0, 0)),
        out_shape=jax.ShapeDtypeStruct((n * nk, kk, kk), jnp.float32),
    )(rank3, ar)
    A_out = ao.reshape(n, nk, kk, kk)

    return x_out, A_out


# speedup vs baseline: 2.4516x; 1.2923x over previous
import functools

import jax
import jax.numpy as jnp
from jax.experimental import pallas as pl
from jax.experimental.pallas import tpu as pltpu

_HI = jax.lax.Precision.HIGHEST


def _b16(a):
    return a.astype(jnp.bfloat16).astype(jnp.float32)


def _invn_body(p_ref, o_ref):
    pb = p_ref[...]
    o_ref[0, 0] = jax.lax.rsqrt(jnp.maximum(jnp.sum(pb * pb), 1e-12))


def _score_body(invn_ref, p_ref, x_ref, y_ref):
    ctb = pl.program_id(1)

    @pl.when(ctb == 0)
    def _init():
        y_ref[...] = jnp.zeros_like(y_ref)

    xb = _b16(x_ref[0])
    pnb = _b16(p_ref[...] * invn_ref[0, 0])
    y_ref[0] += jnp.sum(xb * pnb, axis=0, keepdims=True)


def _select_body(y_ref, yt_ref, rank_ref, sigy_ref):
    n = pl.program_id(0)
    nn = pl.num_programs(0)
    v = y_ref.shape[2]

    yrow = y_ref[0]
    en = (jax.lax.broadcasted_iota(jnp.int32, (nn, 1), 0) == n).astype(jnp.float32)
    ycol = jax.lax.dot_general(yt_ref[...], en, (((1,), (0,)), ((), ())),
                               preferred_element_type=jnp.float32, precision=_HI)

    gt = ycol > yrow
    eq = ycol == yrow
    iw = jax.lax.broadcasted_iota(jnp.int32, (v, v), 0)
    iv = jax.lax.broadcasted_iota(jnp.int32, (v, v), 1)
    m = jnp.logical_or(gt, jnp.logical_and(eq, iw < iv))
    rank_ref[0] = jnp.sum(m.astype(jnp.float32), axis=0, keepdims=True)
    sigy_ref[0] = jax.nn.sigmoid(yrow)


def _xgather_body(rank_ref, sigy_ref, x_ref, o_ref, *, kk):
    v = rank_ref.shape[2]
    rkrow = rank_ref[0]
    sgrow = sigy_ref[0]
    jj = jax.lax.broadcasted_iota(jnp.int32, (kk, v), 0).astype(jnp.float32)
    st = (jj == rkrow).astype(jnp.float32)
    sgate = jax.lax.dot_general(sgrow, st, (((1,), (1,)), ((), ())),
                                preferred_element_type=jnp.float32, precision=_HI)
    xf = x_ref[0]
    xhi = xf.astype(jnp.bfloat16)
    xlo = (xf - xhi.astype(jnp.float32)).astype(jnp.bfloat16)
    stb = st.astype(jnp.bfloat16)
    nt = (((1,), (1,)), ((), ()))
    acc = jax.lax.dot_general(xhi, stb, nt, preferred_element_type=jnp.float32)
    acc += jax.lax.dot_general(xlo, stb, nt, preferred_element_type=jnp.float32)
    o_ref[0] = acc * sgate


def _a_body(rank_ref, a_ref, o_ref, *, kk):
    v = rank_ref.shape[2]
    nk = a_ref.shape[1]
    rkrow = rank_ref[0]
    jj = jax.lax.broadcasted_iota(jnp.int32, (kk, v), 0).astype(jnp.float32)
    st = (jj == rkrow).astype(jnp.bfloat16)
    for k in range(nk):
        a = a_ref[0, k].astype(jnp.bfloat16)
        u = jax.lax.dot_general(st, a, (((1,), (0,)), ((), ())),
                                preferred_element_type=jnp.float32)
        asel = jax.lax.dot_general(a, st, (((1,), (1,)), ((), ())),
                                   preferred_element_type=jnp.float32)
        o_ref[0, k] = jax.lax.dot_general(
            u.astype(jnp.bfloat16), asel.astype(jnp.bfloat16),
            (((1,), (0,)), ((), ())), preferred_element_type=jnp.float32)


@jax.jit
def kernel(x, A, p):
    n, c, t, v = x.shape
    ct = c * t
    kk = v // 2
    nk = A.shape[1]

    xr = x.reshape(n, ct, v)
    pc = p.reshape(ct, 1)

    invn = pl.pallas_call(
        _invn_body,
        out_specs=pl.BlockSpec(memory_space=pltpu.SMEM),
        out_shape=jax.ShapeDtypeStruct((1, 1), jnp.float32),
    )(p.reshape(64, ct // 64))

    blk1 = 8192
    y3 = pl.pallas_call(
        _score_body,
        grid=(n, ct // blk1),
        in_specs=[
            pl.BlockSpec(memory_space=pltpu.SMEM),
            pl.BlockSpec((blk1, 1), lambda i, j: (j, 0)),
            pl.BlockSpec((1, blk1, v), lambda i, j: (i, j, 0)),
        ],
        out_specs=pl.BlockSpec((1, 1, v), lambda i, j: (i, 0, 0)),
        out_shape=jax.ShapeDtypeStruct((n, 1, v), jnp.float32),
    )(invn, pc, xr)

    yt = y3.reshape(n, v).T

    rank3, sigy3 = pl.pallas_call(
        _select_body,
        grid=(n,),
        in_specs=[
            pl.BlockSpec((1, 1, v), lambda i: (i, 0, 0)),
            pl.BlockSpec((v, n), lambda i: (0, 0)),
        ],
        out_specs=[
            pl.BlockSpec((1, 1, v), lambda i: (i, 0, 0)),
            pl.BlockSpec((1, 1, v), lambda i: (i, 0, 0)),
        ],
        out_shape=[
            jax.ShapeDtypeStruct((n, 1, v), jnp.float32),
            jax.ShapeDtypeStruct((n, 1, v), jnp.float32),
        ],
    )(y3, yt)

    blk2 = 8192
    xo = pl.pallas_call(
        functools.partial(_xgather_body, kk=kk),
        grid=(n, ct // blk2),
        in_specs=[
            pl.BlockSpec((1, 1, v), lambda i, j: (i, 0, 0)),
            pl.BlockSpec((1, 1, v), lambda i, j: (i, 0, 0)),
            pl.BlockSpec((1, blk2, v), lambda i, j: (i, j, 0)),
        ],
        out_specs=pl.BlockSpec((1, blk2, kk), lambda i, j: (i, j, 0)),
        out_shape=jax.ShapeDtypeStruct((n, ct, kk), jnp.float32),
    )(rank3, sigy3, xr)
    x_out = xo.reshape(n, c, t, kk)

    A_out = pl.pallas_call(
        functools.partial(_a_body, kk=kk),
        grid=(n,),
        in_specs=[
            pl.BlockSpec((1, 1, v), lambda g: (g, 0, 0)),
            pl.BlockSpec((1, nk, v, v), lambda g: (g, 0, 0, 0)),
        ],
        out_specs=pl.BlockSpec((1, nk, kk, kk), lambda g: (g, 0, 0, 0)),
        out_shape=jax.ShapeDtypeStruct((n, nk, kk, kk), jnp.float32),
    )(rank3, A)

    return x_out, A_out


# speedup vs baseline: 2.7062x; 1.1039x over previous
import functools

import jax
import jax.numpy as jnp
from jax.experimental import pallas as pl
from jax.experimental.pallas import tpu as pltpu

_HI = jax.lax.Precision.HIGHEST
_NT = (((1,), (1,)), ((), ()))
_NN = (((1,), (0,)), ((), ()))


def _b16(a):
    return a.astype(jnp.bfloat16).astype(jnp.float32)


def _score_sel_body(p_ref, x_ref, rank_ref, sigy_ref):
    v = x_ref.shape[2]

    pb = p_ref[...]
    invn = jax.lax.rsqrt(jnp.maximum(jnp.sum(pb * pb), 1e-12))
    pnb = _b16(pb * invn)
    xb = _b16(x_ref[0])
    yrow = jnp.sum(xb * pnb, axis=0, keepdims=True)

    iw = jax.lax.broadcasted_iota(jnp.int32, (v, v), 0)
    iv = jax.lax.broadcasted_iota(jnp.int32, (v, v), 1)
    ident = (iw == iv).astype(jnp.float32)
    ycol = jax.lax.dot_general(ident, yrow, _NT,
                               preferred_element_type=jnp.float32, precision=_HI)

    gt = ycol > yrow
    eq = ycol == yrow
    m = jnp.logical_or(gt, jnp.logical_and(eq, iw < iv))
    rank_ref[0] = jnp.sum(m.astype(jnp.float32), axis=0, keepdims=True)
    sigy_ref[0] = jax.nn.sigmoid(yrow)


def _gather_body(rank_ref, sigy_ref, x_ref, a_ref, xo_ref, ao_ref, *, kk):
    v = rank_ref.shape[2]
    nk = a_ref.shape[1]
    rkrow = rank_ref[0]
    sgrow = sigy_ref[0]
    jj = jax.lax.broadcasted_iota(jnp.int32, (kk, v), 0).astype(jnp.float32)
    stf = (jj == rkrow).astype(jnp.float32)
    sgate = jax.lax.dot_general(sgrow, stf, _NT,
                                preferred_element_type=jnp.float32, precision=_HI)
    stb = stf.astype(jnp.bfloat16)

    xf = x_ref[0]
    xhi = xf.astype(jnp.bfloat16)
    xlo = (xf - xhi.astype(jnp.float32)).astype(jnp.bfloat16)
    acc = jax.lax.dot_general(xhi, stb, _NT, preferred_element_type=jnp.float32)
    acc += jax.lax.dot_general(xlo, stb, _NT, preferred_element_type=jnp.float32)
    xo_ref[0] = acc * sgate

    for k in range(nk):
        a = a_ref[0, k].astype(jnp.bfloat16)
        u = jax.lax.dot_general(stb, a, _NN,
                                preferred_element_type=jnp.float32)
        asel = jax.lax.dot_general(a, stb, _NT,
                                   preferred_element_type=jnp.float32)
        ao_ref[0, k] = jax.lax.dot_general(
            u.astype(jnp.bfloat16), asel.astype(jnp.bfloat16), _NN,
            preferred_element_type=jnp.float32)


@jax.jit
def kernel(x, A, p):
    n, c, t, v = x.shape
    ct = c * t
    kk = v // 2
    nk = A.shape[1]

    xr = x.reshape(n, ct, v)
    pc = p.reshape(ct, 1)

    rank3, sigy3 = pl.pallas_call(
        _score_sel_body,
        grid=(n,),
        in_specs=[
            pl.BlockSpec((ct, 1), lambda i: (0, 0)),
            pl.BlockSpec((1, ct, v), lambda i: (i, 0, 0)),
        ],
        out_specs=[
            pl.BlockSpec((1, 1, v), lambda i: (i, 0, 0)),
            pl.BlockSpec((1, 1, v), lambda i: (i, 0, 0)),
        ],
        out_shape=[
            jax.ShapeDtypeStruct((n, 1, v), jnp.float32),
            jax.ShapeDtypeStruct((n, 1, v), jnp.float32),
        ],
    )(pc, xr)

    xo, A_out = pl.pallas_call(
        functools.partial(_gather_body, kk=kk),
        grid=(n,),
        in_specs=[
            pl.BlockSpec((1, 1, v), lambda i: (i, 0, 0)),
            pl.BlockSpec((1, 1, v), lambda i: (i, 0, 0)),
            pl.BlockSpec((1, ct, v), lambda i: (i, 0, 0)),
            pl.BlockSpec((1, nk, v, v), lambda i: (i, 0, 0, 0)),
        ],
        out_specs=[
            pl.BlockSpec((1, ct, kk), lambda i: (i, 0, 0)),
            pl.BlockSpec((1, nk, kk, kk), lambda i: (i, 0, 0, 0)),
        ],
        out_shape=[
            jax.ShapeDtypeStruct((n, ct, kk), jnp.float32),
            jax.ShapeDtypeStruct((n, nk, kk, kk), jnp.float32),
        ],
    )(rank3, sigy3, xr, A)

    return xo.reshape(n, c, t, kk), A_out
